# baseline (speedup 1.0000x reference)
import functools

import jax
import jax.numpy as jnp
from jax import lax
from jax.experimental import pallas as pl
from jax.experimental.pallas import tpu as pltpu
from jax.experimental.pallas import tpu_sc as plsc

V = 10000
D = 128
B = 320000
L = 16
NC = 2
NS = 16
NW = NC * NS
BPW = B // NW
C = 40
N = BPW // C
NBUF = 4
UR = 4

_mesh = plsc.VectorSubcoreMesh(core_axis_name="c", subcore_axis_name="s")

W = D // 2

_scratch = (
    [pltpu.VMEM((C,), jnp.int32) for _ in range(4)]
    + [pltpu.VMEM((2, C, W), jnp.int32) for _ in range(NBUF)]
    + [pltpu.VMEM((C, D), jnp.float32) for _ in range(NBUF)]
    + [pltpu.VMEM_SHARED((V, W), jnp.int32)]
    + [pltpu.SemaphoreType.DMA for _ in range(2 + 2 * NBUF)]
)


@functools.partial(
    pl.kernel,
    mesh=_mesh,
    out_type=jax.ShapeDtypeStruct((B, D), jnp.float32),
    scratch_types=_scratch,
    compiler_params=pltpu.CompilerParams(use_tc_tiling_on_sc=False),
)
def _product_tuple(x_hbm, idx0_hbm, idx1_hbm, out_hbm, *scr):
    islot = ((scr[0], scr[1]), (scr[2], scr[3]))
    rows = scr[4:4 + NBUF]
    ov = scr[4 + NBUF:4 + 2 * NBUF]
    xs = scr[4 + 2 * NBUF]
    isem = scr[5 + 2 * NBUF:7 + 2 * NBUF]
    gsem = scr[7 + 2 * NBUF:7 + 3 * NBUF]
    wsem = scr[7 + 3 * NBUF:7 + 4 * NBUF]

    sid = lax.axis_index("s")
    wid = sid * NC + lax.axis_index("c")
    base = pl.multiple_of(wid * BPW, 8)

    rows_per_sub = 624
    pltpu.sync_copy(x_hbm.at[pl.ds(sid * rows_per_sub, rows_per_sub)],
                    xs.at[pl.ds(sid * rows_per_sub, rows_per_sub)])

    @pl.when(sid == 0)
    def _stage_tail():
        tail = NS * rows_per_sub
        pltpu.sync_copy(x_hbm.at[pl.ds(tail, V - tail)],
                        xs.at[pl.ds(tail, V - tail)])

    plsc.subcore_barrier()

    def off_of(c):
        return pl.multiple_of(base + c * C, 8)

    def issue_idx(c, p):
        off = off_of(c)
        pltpu.async_copy(idx0_hbm.at[pl.ds(off, C)], islot[p][0], isem[p])
        pltpu.async_copy(idx1_hbm.at[pl.ds(off, C)], islot[p][1], isem[p])

    def wait_idx(p):
        pltpu.make_async_copy(idx0_hbm.at[pl.ds(0, C)], islot[p][0], isem[p]).wait()
        pltpu.make_async_copy(idx1_hbm.at[pl.ds(0, C)], islot[p][1], isem[p]).wait()

    def issue_gather(p, b):
        pltpu.async_copy(xs.at[islot[p][0]], rows[b].at[0], gsem[b])
        pltpu.async_copy(xs.at[islot[p][1]], rows[b].at[1], gsem[b])

    def wait_gather(b):
        pltpu.make_async_copy(xs.at[islot[0][0]], rows[b].at[0], gsem[b]).wait()
        pltpu.make_async_copy(xs.at[islot[0][1]], rows[b].at[1], gsem[b]).wait()

    _HI = jnp.int32(-65536)

    def compute(b):
        r, o = rows[b], ov[b]

        def row_body(t, carry):
            for u in range(UR):
                rr = t * UR + u
                for g in range(D // 32):
                    sl = pl.ds(g * L, L)
                    ai = r[0, rr, sl]
                    bi = r[1, rr, sl]
                    alo = lax.bitcast_convert_type(ai << 16, jnp.float32)
                    ahi = lax.bitcast_convert_type(ai & _HI, jnp.float32)
                    blo = lax.bitcast_convert_type(bi << 16, jnp.float32)
                    bhi = lax.bitcast_convert_type(bi & _HI, jnp.float32)
                    o[rr, pl.ds(g * 32, L)] = alo * blo
                    o[rr, pl.ds(g * 32 + L, L)] = ahi * bhi
            return carry

        lax.fori_loop(0, C // UR, row_body, 0)

    def issue_wb(c, b):
        pltpu.async_copy(ov[b], out_hbm.at[pl.ds(off_of(c), C)], wsem[b])

    def wait_wb(b):
        pltpu.make_async_copy(ov[b], out_hbm.at[pl.ds(0, C)], wsem[b]).wait()

    def step(c, b, p, has_next=True, idx_ahead=True, drain_wb=True):
        wait_gather(b)
        if idx_ahead:
            issue_idx(c + 2, p)
        if has_next:
            wait_idx(1 - p)
            issue_gather(1 - p, (b + 1) % NBUF)
        if drain_wb:
            wait_wb(b)
        compute(b)
        issue_wb(c, b)

    issue_idx(0, 0)
    issue_idx(1, 1)
    wait_idx(0)
    issue_gather(0, 0)

    for c in range(NBUF):
        step(c, c % NBUF, c % 2, drain_wb=False)

    RSTEADY = (N - NBUF - 2) // 4

    def steady(i, carry):
        c0 = NBUF + i * 4
        for j in range(4):
            step(c0 + j, j, j % 2)
        return carry

    lax.fori_loop(0, RSTEADY, steady, 0)

    for c in range(NBUF + 4 * RSTEADY, N):
        step(c, c % NBUF, c % 2,
             has_next=(c + 1 <= N - 1),
             idx_ahead=(c + 2 <= N - 1))

    for b in range(NBUF):
        wait_wb(b)


def _prep_table(X):
    Xp = X.reshape(V, D // 32, 2, 16).swapaxes(2, 3).reshape(V, D)
    Xbf = Xp.astype(jnp.bfloat16)
    return lax.bitcast_convert_type(Xbf.reshape(V, W, 2), jnp.int32)


def kernel(X, adj_t, tuples_coo):
    del adj_t
    return _product_tuple(_prep_table(X), tuples_coo[0], tuples_coo[1])


# speedup vs baseline: 1.5683x; 1.5683x over previous
import functools

import jax
import jax.numpy as jnp
from jax import lax
from jax.experimental import pallas as pl
from jax.experimental.pallas import tpu as pltpu
from jax.experimental.pallas import tpu_sc as plsc

V = 10000
D = 128
B = 320000
L = 16
NC = 2
NS = 16
NW = NC * NS
BPW = B // NW
C = 40
N = BPW // C
NBUF = 2
UR = 4

_mesh = plsc.VectorSubcoreMesh(core_axis_name="c", subcore_axis_name="s")

_scratch = (
    [pltpu.VMEM((BPW,), jnp.int32) for _ in range(2)]
    + [pltpu.VMEM((2, C, D), jnp.float32) for _ in range(NBUF)]
    + [pltpu.VMEM_SHARED((V, D), jnp.float32)]
    + [pltpu.SemaphoreType.DMA for _ in range(2 * NBUF)]
)


@functools.partial(
    pl.kernel,
    mesh=_mesh,
    out_type=jax.ShapeDtypeStruct((B, D), jnp.float32),
    scratch_types=_scratch,
)
def _product_tuple(x_hbm, idx0_hbm, idx1_hbm, out_hbm, *scr):
    idxa = scr[0:2]
    rows = scr[2:2 + NBUF]
    xs = scr[2 + NBUF]
    gsem = scr[3 + NBUF:3 + 2 * NBUF]
    wsem = scr[3 + 2 * NBUF:3 + 3 * NBUF]

    sid = lax.axis_index("s")
    wid = sid * NC + lax.axis_index("c")
    base = pl.multiple_of(wid * BPW, 8)

    rows_per_sub = 624
    pltpu.sync_copy(x_hbm.at[pl.ds(sid * rows_per_sub, rows_per_sub)],
                    xs.at[pl.ds(sid * rows_per_sub, rows_per_sub)])

    @pl.when(sid == 0)
    def _stage_tail():
        tail = NS * rows_per_sub
        pltpu.sync_copy(x_hbm.at[pl.ds(tail, V - tail)],
                        xs.at[pl.ds(tail, V - tail)])

    pltpu.sync_copy(idx0_hbm.at[pl.ds(base, BPW)], idxa[0])
    pltpu.sync_copy(idx1_hbm.at[pl.ds(base, BPW)], idxa[1])

    plsc.subcore_barrier()

    def off_of(c):
        return pl.multiple_of(base + c * C, 8)

    def issue_gather(c, b):
        coff = pl.multiple_of(c * C, 8)
        pltpu.async_copy(xs.at[idxa[0].at[pl.ds(coff, C)]], rows[b].at[0], gsem[b])
        pltpu.async_copy(xs.at[idxa[1].at[pl.ds(coff, C)]], rows[b].at[1], gsem[b])

    def wait_gather(b):
        pltpu.make_async_copy(xs.at[idxa[0].at[pl.ds(0, C)]], rows[b].at[0], gsem[b]).wait()
        pltpu.make_async_copy(xs.at[idxa[1].at[pl.ds(0, C)]], rows[b].at[1], gsem[b]).wait()

    def compute(b):
        r = rows[b]

        def row_body(t, carry):
            for u in range(UR):
                rr = t * UR + u
                for j in range(D // L):
                    s = pl.ds(j * L, L)
                    r[0, rr, s] = r[0, rr, s] * r[1, rr, s]
            return carry

        lax.fori_loop(0, C // UR, row_body, 0)

    def issue_wb(c, b):
        pltpu.async_copy(rows[b].at[0], out_hbm.at[pl.ds(off_of(c), C)], wsem[b])

    def wait_wb(b):
        pltpu.make_async_copy(rows[b].at[0], out_hbm.at[pl.ds(0, C)], wsem[b]).wait()

    def step(c, b, has_next=True, drain_wb=True):
        wait_gather(b)
        if has_next:
            if drain_wb:
                wait_wb(1 - b)
            issue_gather(c + 1, 1 - b)
        compute(b)
        issue_wb(c, b)

    issue_gather(0, 0)

    for c in range(NBUF):
        step(c, c % NBUF, drain_wb=(c >= NBUF - 1))

    RSTEADY = (N - NBUF - 1) // 2

    def steady(i, carry):
        c0 = NBUF + i * 2
        for j in range(2):
            step(c0 + j, j)
        return carry

    lax.fori_loop(0, RSTEADY, steady, 0)

    for c in range(NBUF + 2 * RSTEADY, N):
        step(c, c % NBUF, has_next=(c + 1 <= N - 1))

    for b in range(NBUF):
        wait_wb(b)


def kernel(X, adj_t, tuples_coo):
    del adj_t
    return _product_tuple(X, tuples_coo[0], tuples_coo[1])
